# baseline (speedup 1.0000x reference)
import functools

import jax
import jax.numpy as jnp
from jax.experimental import pallas as pl
from jax.experimental.pallas import tpu as pltpu

_MT = 512


def _chamfer_body(nj, scale, xa_ref, ya_ref, out_ref, minl_ref):
    b = pl.program_id(0)
    j = pl.program_id(1)

    d = jax.lax.dot_general(
        xa_ref[0], ya_ref[0],
        dimension_numbers=(((1,), (0,)), ((), ())),
        preferred_element_type=jnp.float32,
    )

    @pl.when(j == 0)
    def _():
        minl_ref[...] = d

    @pl.when(j > 0)
    def _():
        minl_ref[...] = jnp.minimum(minl_ref[...], d)

    @pl.when((b == 0) & (j == 0))
    def _():
        out_ref[...] = jnp.zeros((1, 1), jnp.float32)

    minr = jnp.min(d, axis=0, keepdims=True)
    out_ref[...] += jnp.sum(minr, axis=1, keepdims=True) * scale

    @pl.when(j == nj - 1)
    def _():
        minl = jnp.min(minl_ref[...], axis=1, keepdims=True)
        out_ref[...] += jnp.sum(minl, axis=0, keepdims=True) * scale


def _augment(x, y):
    f32 = jnp.float32
    bf16 = jnp.bfloat16
    B, N, _ = x.shape

    xx = jnp.sum(x * x, axis=2, keepdims=True)
    yy = jnp.sum(y * y, axis=2, keepdims=True)
    ax = xx.astype(bf16)
    bx = (xx - ax.astype(f32)).astype(bf16)
    ay = yy.astype(bf16)
    by = (yy - ay.astype(f32)).astype(bf16)

    ones = jnp.ones_like(ax)
    zeros = jnp.zeros_like(ax)
    xa = jnp.concatenate(
        [x.astype(bf16), ax, bx, ones, ones, zeros], axis=2)
    ya = jnp.concatenate(
        [(-2.0 * y.astype(bf16).astype(f32)).astype(bf16),
         ones, ones, ay, by, zeros], axis=2)
    return xa, jnp.swapaxes(ya, 1, 2)


def kernel(x, y):
    B, N, D = x.shape
    M = y.shape[1]
    nj = M // _MT
    scale = 1.0 / (B * N)

    xa, yat = _augment(x, y)

    body = functools.partial(_chamfer_body, nj, scale)

    out = pl.pallas_call(
        body,
        grid=(B, nj),
        in_specs=[
            pl.BlockSpec((1, N, 8), lambda b, j: (b, 0, 0)),
            pl.BlockSpec((1, 8, _MT), lambda b, j: (b, 0, j)),
        ],
        out_specs=pl.BlockSpec((1, 1), lambda b, j: (0, 0)),
        out_shape=jax.ShapeDtypeStruct((1, 1), jnp.float32),
        scratch_shapes=[pltpu.VMEM((N, _MT), jnp.float32)],
    )(xa, yat)
    return out[0, 0]


# speedup vs baseline: 1.4071x; 1.4071x over previous
import functools

import jax
import jax.numpy as jnp
from jax.experimental import pallas as pl


def _chamfer_body(scale, xa_ref, ya_ref, out_ref):
    b = pl.program_id(0)

    d = jax.lax.dot_general(
        xa_ref[0], ya_ref[0],
        dimension_numbers=(((1,), (0,)), ((), ())),
        preferred_element_type=jnp.float32,
    )

    M = d.shape[1]

    chunks = [d[:, k:k + 128] for k in range(0, M, 128)]
    while len(chunks) > 1:
        chunks = [jnp.minimum(chunks[i], chunks[i + 1])
                  for i in range(0, len(chunks), 2)]
    minl = jnp.min(chunks[0], axis=1, keepdims=True)
    suml = jnp.sum(minl, axis=0, keepdims=True)

    minr = jnp.min(d, axis=0, keepdims=True)
    sumr = jnp.sum(minr, axis=1, keepdims=True)

    @pl.when(b == 0)
    def _():
        out_ref[...] = jnp.zeros((1, 1), jnp.float32)

    out_ref[...] += (suml + sumr) * scale


def _augment(x, y):
    f32 = jnp.float32
    bf16 = jnp.bfloat16

    xx = jnp.sum(x * x, axis=2, keepdims=True)
    yy = jnp.sum(y * y, axis=2, keepdims=True)
    ax = xx.astype(bf16)
    bx = (xx - ax.astype(f32)).astype(bf16)
    ay = yy.astype(bf16)
    by = (yy - ay.astype(f32)).astype(bf16)

    ones = jnp.ones_like(ax)
    zeros = jnp.zeros_like(ax)
    xa = jnp.concatenate(
        [x.astype(bf16), ax, bx, ones, ones, zeros], axis=2)
    ya = jnp.concatenate(
        [(-2.0 * y.astype(bf16).astype(f32)).astype(bf16),
         ones, ones, ay, by, zeros], axis=2)
    return xa, jnp.swapaxes(ya, 1, 2)


def kernel(x, y):
    B, N, D = x.shape
    M = y.shape[1]
    scale = 1.0 / (B * N)

    xa, yat = _augment(x, y)

    body = functools.partial(_chamfer_body, scale)

    out = pl.pallas_call(
        body,
        grid=(B,),
        in_specs=[
            pl.BlockSpec((1, N, 8), lambda b: (b, 0, 0)),
            pl.BlockSpec((1, 8, M), lambda b: (b, 0, 0)),
        ],
        out_specs=pl.BlockSpec((1, 1), lambda b: (0, 0)),
        out_shape=jax.ShapeDtypeStruct((1, 1), jnp.float32),
    )(xa, yat)
    return out[0, 0]


# speedup vs baseline: 3.0981x; 2.2017x over previous
import functools

import jax
import jax.numpy as jnp
from jax.experimental import pallas as pl


def _chamfer_body(scale, x_ref, y_ref, out_ref):
    b = pl.program_id(0)
    f32 = jnp.float32
    bf16 = jnp.bfloat16

    xb = x_ref[0]
    yb = y_ref[0]

    xx = jnp.sum(xb * xb, axis=1, keepdims=True)
    yy = jnp.sum(yb * yb, axis=1, keepdims=True)
    ax = xx.astype(bf16)
    bx = (xx - ax.astype(f32)).astype(bf16)
    ay = yy.astype(bf16)
    by = (yy - ay.astype(f32)).astype(bf16)

    ones = jnp.ones_like(ax)
    zeros = jnp.zeros_like(ax)
    xa = jnp.concatenate(
        [xb.astype(bf16), ax, bx, ones, ones, zeros], axis=1)
    ya = jnp.concatenate(
        [(-2.0 * yb.astype(bf16).astype(f32)).astype(bf16),
         ones, ones, ay, by, zeros], axis=1)

    d = jax.lax.dot_general(
        xa, ya,
        dimension_numbers=(((1,), (1,)), ((), ())),
        preferred_element_type=jnp.float32,
    )

    M = d.shape[1]

    chunks = [d[:, k:k + 128] for k in range(0, M, 128)]
    while len(chunks) > 1:
        chunks = [jnp.minimum(chunks[i], chunks[i + 1])
                  for i in range(0, len(chunks), 2)]
    minl = jnp.min(chunks[0], axis=1, keepdims=True)
    suml = jnp.sum(minl, axis=0, keepdims=True)

    minr = jnp.min(d, axis=0, keepdims=True)
    sumr = jnp.sum(minr, axis=1, keepdims=True)

    @pl.when(b == 0)
    def _():
        out_ref[...] = jnp.zeros((1, 1), jnp.float32)

    out_ref[...] += (suml + sumr) * scale


def kernel(x, y):
    B, N, D = x.shape
    M = y.shape[1]
    scale = 1.0 / (B * N)

    body = functools.partial(_chamfer_body, scale)

    out = pl.pallas_call(
        body,
        grid=(B,),
        in_specs=[
            pl.BlockSpec((1, N, D), lambda b: (b, 0, 0)),
            pl.BlockSpec((1, M, D), lambda b: (b, 0, 0)),
        ],
        out_specs=pl.BlockSpec((1, 1), lambda b: (0, 0)),
        out_shape=jax.ShapeDtypeStruct((1, 1), jnp.float32),
    )(x, y)
    return out[0, 0]
